# Initial kernel scaffold; baseline (speedup 1.0000x reference)
import jax
import jax.numpy as jnp
from jax.experimental import pallas as pl


def kernel(x, router_weight, shared_gate, shared_up, shared_down, gate_s, up_s, down_s, gate_w, up_w, down_w):
    raise NotImplementedError("write your pallas kernel here")


# speedup vs baseline: 1.4445x; 1.4445x over previous
import functools

import jax
import jax.numpy as jnp
from jax.experimental import pallas as pl
from jax.experimental.pallas import tpu as pltpu

B, L, D = 1, 2048, 1024
I = 2816
E = 8
TOP_K = 2

_T_BLK = 512


def _router_body(x_ref, rw_ref, w_ref):
    x = x_ref[...]
    logits = jax.lax.dot_general(
        x, rw_ref[...], (((1,), (1,)), ((), ())),
        preferred_element_type=jnp.float32)
    probs = jax.nn.softmax(logits, axis=-1)
    lane = jax.lax.broadcasted_iota(jnp.int32, probs.shape, 1)
    a1 = jnp.argmax(probs, axis=-1)
    sel1 = lane == a1[:, None]
    m1 = jnp.max(probs, axis=-1, keepdims=True)
    masked = jnp.where(sel1, -jnp.inf, probs)
    a2 = jnp.argmax(masked, axis=-1)
    sel2 = lane == a2[:, None]
    m2 = jnp.max(masked, axis=-1, keepdims=True)
    w = jnp.where(sel1, m1, 0.0) + jnp.where(sel2, m2, 0.0)
    w_ref[...] = w / (m1 + m2)


def _shared_body(x_ref, sg_ref, su_ref, sd_ref, out_ref):
    x = x_ref[...]
    g = jax.lax.dot_general(x, sg_ref[...], (((1,), (1,)), ((), ())),
                            preferred_element_type=jnp.float32)
    u = jax.lax.dot_general(x, su_ref[...], (((1,), (1,)), ((), ())),
                            preferred_element_type=jnp.float32)
    h = (jax.nn.silu(g) * u).astype(jnp.bfloat16)
    out_ref[...] = jax.lax.dot_general(
        h, sd_ref[...], (((1,), (1,)), ((), ())),
        preferred_element_type=jnp.float32)


def _experts_body(w_ref, gs_ref, us_ref, ds_ref, x_ref, gw_ref, uw_ref,
                  dw_ref, shared_ref, out_ref):
    e = pl.program_id(0)
    t = pl.program_id(1)
    xb = x_ref[...]
    gw = gw_ref[0].astype(jnp.bfloat16)
    uw = uw_ref[0].astype(jnp.bfloat16)
    dw = dw_ref[0].astype(jnp.bfloat16)
    g = jax.lax.dot_general(xb, gw, (((1,), (1,)), ((), ())),
                            preferred_element_type=jnp.float32) * gs_ref[e]
    u = jax.lax.dot_general(xb, uw, (((1,), (1,)), ((), ())),
                            preferred_element_type=jnp.float32) * us_ref[e]
    h = (jax.nn.silu(g) * u).astype(jnp.bfloat16)
    o = jax.lax.dot_general(h, dw, (((1,), (1,)), ((), ())),
                            preferred_element_type=jnp.float32) * ds_ref[e]
    wmat = w_ref[...]
    lane = jax.lax.broadcasted_iota(jnp.int32, wmat.shape, 1)
    wcol = jnp.sum(jnp.where(lane == e, wmat, 0.0), axis=1)
    contrib = wcol[:, None] * o
    rows = pl.ds(t * _T_BLK, _T_BLK)

    @pl.when(e == 0)
    def _():
        out_ref[rows, :] = contrib + shared_ref[...]

    @pl.when(e > 0)
    def _():
        out_ref[rows, :] = out_ref[rows, :] + contrib


@jax.jit
def kernel(x, router_weight, shared_gate, shared_up, shared_down, gate_s,
           up_s, down_s, gate_w, up_w, down_w):
    xf = x.reshape(-1, D)
    xb = xf.astype(jnp.bfloat16)
    nt = L // _T_BLK

    w = pl.pallas_call(
        _router_body,
        out_shape=jax.ShapeDtypeStruct((L, E), jnp.float32),
        in_specs=[pl.BlockSpec((L, D), lambda: (0, 0)),
                  pl.BlockSpec((E, D), lambda: (0, 0))],
        out_specs=pl.BlockSpec((L, E), lambda: (0, 0)),
    )(xf, router_weight)

    shared_out = pl.pallas_call(
        _shared_body,
        grid=(nt,),
        out_shape=jax.ShapeDtypeStruct((L, D), jnp.float32),
        in_specs=[pl.BlockSpec((_T_BLK, D), lambda t: (t, 0)),
                  pl.BlockSpec((I, D), lambda t: (0, 0)),
                  pl.BlockSpec((I, D), lambda t: (0, 0)),
                  pl.BlockSpec((D, I), lambda t: (0, 0))],
        out_specs=pl.BlockSpec((_T_BLK, D), lambda t: (t, 0)),
        compiler_params=pltpu.CompilerParams(
            dimension_semantics=("arbitrary",)),
    )(xb, shared_gate.astype(jnp.bfloat16), shared_up.astype(jnp.bfloat16),
      shared_down.astype(jnp.bfloat16))

    out = pl.pallas_call(
        _experts_body,
        grid=(E, nt),
        out_shape=jax.ShapeDtypeStruct((L, D), jnp.float32),
        in_specs=[
            pl.BlockSpec((_T_BLK, E), lambda e, t: (t, 0)),
            pl.BlockSpec(memory_space=pltpu.SMEM),
            pl.BlockSpec(memory_space=pltpu.SMEM),
            pl.BlockSpec(memory_space=pltpu.SMEM),
            pl.BlockSpec((_T_BLK, D), lambda e, t: (t, 0)),
            pl.BlockSpec((1, I, D), lambda e, t: (e, 0, 0)),
            pl.BlockSpec((1, I, D), lambda e, t: (e, 0, 0)),
            pl.BlockSpec((1, D, I), lambda e, t: (e, 0, 0)),
            pl.BlockSpec((_T_BLK, D), lambda e, t: (t, 0)),
        ],
        out_specs=pl.BlockSpec((L, D), lambda e, t: (0, 0)),
        compiler_params=pltpu.CompilerParams(
            dimension_semantics=("arbitrary", "arbitrary")),
    )(w, gate_s, up_s, down_s, xb, gate_w, up_w, down_w, shared_out)

    return out.reshape(x.shape).astype(x.dtype)
